# baseline (speedup 1.0000x reference)
import jax
import jax.numpy as jnp
from jax.experimental import pallas as pl
from jax.experimental.pallas import tpu as pltpu


def _fps_kernel(x_ref, out_ref):
    n = x_ref.shape[1]
    n_center = out_ref.shape[1]
    x = x_ref[...]
    iota = jax.lax.broadcasted_iota(jnp.int32, (1, n), 1)

    def body(i, carry):
        dist, far = carry
        out_ref[0, i] = far
        mask = iota == far
        c = jnp.sum(jnp.where(mask, x, 0.0), axis=1, keepdims=True)
        d = jnp.sum((x - c) ** 2, axis=0, keepdims=True)
        dist = jnp.minimum(dist, d)
        far = jnp.argmax(dist).astype(jnp.int32)
        return dist, far

    dist0 = jnp.full((1, n), 1e10, jnp.float32)
    jax.lax.fori_loop(0, n_center, body, (dist0, jnp.int32(0)))


def fps(xyz, n_center):
    x = jax.lax.stop_gradient(xyz)
    b, n, _ = x.shape
    xt = jnp.transpose(x, (0, 2, 1))
    outs = []
    for bi in range(b):
        out = pl.pallas_call(
            _fps_kernel,
            out_shape=jax.ShapeDtypeStruct((1, n_center), jnp.int32),
            in_specs=[pl.BlockSpec(memory_space=pltpu.VMEM)],
            out_specs=pl.BlockSpec(memory_space=pltpu.SMEM),
        )(xt[bi])
        outs.append(out)
    return jnp.concatenate(outs, axis=0)


def _mlp_chain(h, wb_refs, relu_last=False):
    nl = len(wb_refs)
    for i, (w_ref, b_ref) in enumerate(wb_refs):
        h = jax.lax.dot_general(h, w_ref[...], (((1,), (0,)), ((), ())),
                                preferred_element_type=jnp.float32)
        h = h + b_ref[0:1, :]
        if i < nl - 1 or relu_last:
            h = jnp.maximum(h, 0.0)
    return h


def _mlp_kernel(x_ref, *refs, n_layers):
    wb = [(refs[2 * i], refs[2 * i + 1]) for i in range(n_layers)]
    out_ref = refs[2 * n_layers]
    out_ref[...] = _mlp_chain(x_ref[...], wb)


def pallas_mlp(x, p):
    import functools
    b, n, _ = x.shape
    t_blk = 256
    n_pad = ((n + t_blk - 1) // t_blk) * t_blk
    xp = jnp.pad(x, ((0, 0), (0, n_pad - n), (0, 0)))
    args, specs = [], []
    for w, bias in zip(p['W'], p['b']):
        args.append(w)
        specs.append(pl.BlockSpec(w.shape, lambda bi, ti: (0, 0)))
        bb = jnp.broadcast_to(bias[None, :], (8, bias.shape[0]))
        args.append(bb)
        specs.append(pl.BlockSpec(bb.shape, lambda bi, ti: (0, 0)))
    c_out = p['W'][-1].shape[1]
    out = pl.pallas_call(
        functools.partial(_mlp_kernel, n_layers=len(p['W'])),
        grid=(b, n_pad // t_blk),
        in_specs=[pl.BlockSpec((None, t_blk, x.shape[2]),
                               lambda bi, ti: (bi, ti, 0))] + specs,
        out_specs=pl.BlockSpec((None, t_blk, c_out), lambda bi, ti: (bi, ti, 0)),
        out_shape=jax.ShapeDtypeStruct((b, n_pad, c_out), jnp.float32),
    )(xp, *args)
    return out[:, :n]


def _heads_kernel(x_ref, *refs):
    (fw1, fb1, fw2, fb2,
     mw1, mb1, mw2, mb2, mw3, mb3, mw4, mb4,
     aw1, ab1, aw2, ab2, aw3, ab3, aw4, ab4,
     pw1, pb1, pw2, pb2, pw3, pb3, pw4, pb4,
     mad_ref, adj_ref, pt_ref) = refs

    feat = _mlp_chain(x_ref[...], [(fw1, fb1), (fw2, fb2)])

    def head(ws, out_ref, logsm):
        h = _mlp_chain(feat, ws)
        if logsm:
            m = jnp.max(h, axis=1, keepdims=True)
            e = jnp.exp(h - m)
            h = (h - m) - jnp.log(jnp.sum(e, axis=1, keepdims=True))
        out_ref[...] = h

    head([(mw1, mb1), (mw2, mb2), (mw3, mb3), (mw4, mb4)], mad_ref, False)
    head([(aw1, ab1), (aw2, ab2), (aw3, ab3), (aw4, ab4)], adj_ref, True)
    head([(pw1, pb1), (pw2, pb2), (pw3, pb3), (pw4, pb4)], pt_ref, True)


def heads(x, params):
    b, n, _ = x.shape
    t_blk = 256
    chains = [params[k] for k in ('mlp_fea', 'mlp_mad', 'mlp_adj', 'mlp_pt')]
    args, specs = [], []
    for p in chains:
        for w, bias in zip(p['W'], p['b']):
            args.append(w)
            specs.append(pl.BlockSpec(w.shape, lambda bi, ti: (0, 0)))
            bb = jnp.broadcast_to(bias[None, :], (8, bias.shape[0]))
            args.append(bb)
            specs.append(pl.BlockSpec(bb.shape, lambda bi, ti: (0, 0)))
    out_dims = [params['mlp_mad']['W'][-1].shape[1],
                params['mlp_adj']['W'][-1].shape[1],
                params['mlp_pt']['W'][-1].shape[1]]
    outs = pl.pallas_call(
        _heads_kernel,
        grid=(b, n // t_blk),
        in_specs=[pl.BlockSpec((None, t_blk, x.shape[2]),
                               lambda bi, ti: (bi, ti, 0))] + specs,
        out_specs=[pl.BlockSpec((None, t_blk, c), lambda bi, ti: (bi, ti, 0))
                   for c in out_dims],
        out_shape=[jax.ShapeDtypeStruct((b, n, c), jnp.float32)
                   for c in out_dims],
    )(x, *args)
    return outs


def square_distance(src, dst):
    return (jnp.sum(src ** 2, -1)[:, :, None] + jnp.sum(dst ** 2, -1)[:, None, :]
            - 2.0 * jnp.einsum('bnc,bmc->bnm', src, dst))


def index_points(points, idx):
    return jax.vmap(lambda p, i: p[i])(points, idx)


def surface_knn(xyz, n_near):
    x = jax.lax.stop_gradient(xyz)
    d = square_distance(x, x)
    _, idx = jax.lax.top_k(-d, n_near)
    return idx


def point_attention(center_fea, g_fea, p):
    q = center_fea @ p['Wq']
    k = g_fea @ p['Wk']
    v = g_fea @ p['Wv']
    logits = jnp.einsum('bnc,bnkc->bnk', q, k) / jnp.sqrt(jnp.float32(q.shape[-1]))
    attn = jax.nn.softmax(logits, axis=-1)
    return jnp.einsum('bnk,bnkc->bnc', attn, v)


def sa_layer(xyz_cn, fea_cn, p, n_center, n_near):
    xyz = xyz_cn.transpose(0, 2, 1)
    fea = fea_cn.transpose(0, 2, 1)
    idx_all = surface_knn(xyz, n_near)
    fps_idx = fps(xyz, n_center)
    idx = index_points(idx_all, fps_idx)
    center_xyz = index_points(xyz, fps_idx)
    g_xyz = index_points(xyz, idx)
    xyz_rel = g_xyz - center_xyz[:, :, None, :]
    center_fea = index_points(fea, fps_idx)
    g_fea = index_points(fea, idx)
    g_fea = jnp.concatenate([g_fea, xyz_rel], axis=-1)
    new_fea = point_attention(center_fea, g_fea, p)
    return center_xyz.transpose(0, 2, 1), new_fea.transpose(0, 2, 1)


def upsample(xyz1_cn, xyz2_cn, points1_cn, points2_cn, p):
    xyz1 = xyz1_cn.transpose(0, 2, 1)
    xyz2 = xyz2_cn.transpose(0, 2, 1)
    points2 = points2_cn.transpose(0, 2, 1)
    d = square_distance(xyz1, xyz2)
    neg_vals, idx = jax.lax.top_k(-d, 3)
    dists = -neg_vals
    dist_recip = 1.0 / (dists + 1e-8)
    norm = jnp.sum(dist_recip, axis=2, keepdims=True)
    weight = dist_recip / norm
    interpolated = jnp.sum(index_points(points2, idx) * weight[..., None], axis=2)
    points1 = points1_cn.transpose(0, 2, 1)
    new_points = jnp.concatenate([points1, interpolated], axis=-1)
    return pallas_mlp(new_points, p).transpose(0, 2, 1)


def kernel(xyz, params):
    n_points = xyz.shape[1]
    drate = 0.9
    n1 = int(n_points * drate)
    n2 = int(n_points * drate ** 2)
    n3 = int(n_points * drate ** 3)
    x0 = xyz.transpose(0, 2, 1)
    l1_xyz, l1_points = sa_layer(x0, x0, params['sa1'], n1, 50)
    l2_xyz, l2_points = sa_layer(l1_xyz, l1_points, params['sa2'], n2, 75)
    l3_xyz, l3_points = sa_layer(l2_xyz, l2_points, params['sa3'], n3, 100)
    l2_points = upsample(l2_xyz, l3_xyz, l2_points, l3_points, params['fp3'])
    l1_points = upsample(l1_xyz, l2_xyz, l1_points, l2_points, params['fp2'])
    l0_points = upsample(x0, l1_xyz, jnp.concatenate([x0, x0], axis=1), l1_points, params['fp1'])
    mad, adj, pt = heads(l0_points.transpose(0, 2, 1), params)
    return mad, adj, pt
